# Initial kernel scaffold; baseline (speedup 1.0000x reference)
import jax
import jax.numpy as jnp
from jax.experimental import pallas as pl


def kernel(edge_index, feat, W, attn_l, attn_r, bias):
    raise NotImplementedError("write your pallas kernel here")


# speedup vs baseline: 34.2222x; 34.2222x over previous
import functools

import jax
import jax.numpy as jnp
from jax import lax
from jax.experimental import pallas as pl
from jax.experimental.pallas import tpu as pltpu
from jax.experimental.pallas import tpu_sc as plsc

NEG_SLOPE = 0.2
NC = 2
NS = 16
NW = NC * NS
L = 16


def _per_subcore_rows(sid, n_rows, copy_fn):
    rpb = (n_rows // NS) & ~7
    last = n_rows - (NS - 1) * rpb

    @pl.when(sid < NS - 1)
    def _():
        copy_fn(pl.multiple_of(sid * rpb, 8), rpb)

    @pl.when(sid == NS - 1)
    def _():
        copy_fn((NS - 1) * rpb, last)


def _proj_body(x_ref, w_ref, wc_ref, fp_ref, elr_ref):
    x = x_ref[...]
    fp_ref[...] = lax.dot_general(
        x, w_ref[...], (((1,), (1,)), ((), ())),
        preferred_element_type=jnp.float32)
    elr_ref[...] = lax.dot_general(
        x, wc_ref[...], (((1,), (1,)), ((), ())),
        preferred_element_type=jnp.float32)


def _invd_body(d_ref, o_ref):
    d = d_ref[...]
    o_ref[...] = 1.0 / (d[0] + d[1] + 1e-9)


def _merge_body(r_ref, b_ref, o_ref):
    r = r_ref[...]
    o_ref[...] = r[0] + r[1] + b_ref[...]


def _make_scores(N, E, H):
    C = 80
    EPW = E // NW
    NCH = EPW // C
    mesh = plsc.VectorSubcoreMesh(core_axis_name="c", subcore_axis_name="s")

    @functools.partial(
        pl.kernel,
        out_type=[
            jax.ShapeDtypeStruct((E, H), jnp.float32),
            jax.ShapeDtypeStruct((NC, N, H), jnp.float32),
        ],
        mesh=mesh,
        scratch_types=[
            pltpu.VMEM((C,), jnp.int32),
            pltpu.VMEM((C,), jnp.int32),
            pltpu.VMEM((C, 2 * H), jnp.float32),
            pltpu.VMEM((C, 2 * H), jnp.float32),
            pltpu.VMEM((C, H), jnp.float32),
            pltpu.VMEM_SHARED((N, H), jnp.float32),
            pltpu.SemaphoreType.DMA,
            pltpu.SemaphoreType.DMA,
        ],
        compiler_params=pltpu.CompilerParams(needs_layout_passes=False, use_tc_tiling_on_sc=False),
    )
    def scores(elr_hbm, src_hbm, dst_hbm, z_hbm, s_hbm, dpart_hbm,
               isrc, idst, rsrc, rdst, sbuf, dsh, sem1, sem2):
        cid = lax.axis_index("c")
        sid = lax.axis_index("s")
        wid = cid * NS + sid

        def zero_rows(off, size):
            pltpu.sync_copy(z_hbm.at[pl.ds(off, size)],
                            dsh.at[pl.ds(off, size)])

        _per_subcore_rows(sid, N, zero_rows)
        plsc.subcore_barrier()
        iota = lax.iota(jnp.int32, L)
        cols_l = [jnp.full((L,), h, jnp.int32) for h in range(H)]
        cols_r = [jnp.full((L,), H + h, jnp.int32) for h in range(H)]

        def chunk(k, carry):
            base = wid * EPW + k * C
            pltpu.sync_copy(src_hbm.at[pl.ds(base, C)], isrc)
            pltpu.sync_copy(dst_hbm.at[pl.ds(base, C)], idst)
            d1 = pltpu.make_async_copy(elr_hbm.at[isrc], rsrc, sem1)
            d2 = pltpu.make_async_copy(elr_hbm.at[idst], rdst, sem2)
            d1.start()
            d2.start()
            d1.wait()
            d2.wait()
            for g in range(C // L):
                rows = g * L + iota
                for h in range(H):
                    el = plsc.load_gather(rsrc, [rows, cols_l[h]])
                    er = plsc.load_gather(rdst, [rows, cols_r[h]])
                    e = el + er
                    e = jnp.where(e > 0.0, e, NEG_SLOPE * e)
                    plsc.store_scatter(sbuf, [rows, cols_l[h]], jnp.exp(e))
            pltpu.sync_copy(sbuf, s_hbm.at[pl.ds(base, C)])
            pltpu.sync_copy(sbuf, dsh.at[idst], add=True)
            return carry

        lax.fori_loop(0, NCH, chunk, 0)
        plsc.subcore_barrier()

        def export_rows(off, size):
            pltpu.sync_copy(dsh.at[pl.ds(off, size)],
                            dpart_hbm.at[cid, pl.ds(off, size)])

        _per_subcore_rows(sid, N, export_rows)

    return scores


def _make_agg(N, E, H, HD):
    C = 80
    EPW = E // NW
    NCH = EPW // C
    mesh = plsc.VectorSubcoreMesh(core_axis_name="c", subcore_axis_name="s")

    @functools.partial(
        pl.kernel,
        out_type=jax.ShapeDtypeStruct((NC, N, HD), jnp.float32),
        mesh=mesh,
        scratch_types=[
            pltpu.VMEM((C,), jnp.int32),
            pltpu.VMEM((C,), jnp.int32),
            pltpu.VMEM((C, H), jnp.float32),
            pltpu.VMEM((C, H), jnp.float32),
            pltpu.VMEM((C, H), jnp.float32),
            pltpu.VMEM((C, HD), jnp.float32),
            pltpu.VMEM_SHARED((N, H), jnp.float32),
            pltpu.VMEM_SHARED((N, HD), jnp.float32),
            pltpu.SemaphoreType.DMA,
            pltpu.SemaphoreType.DMA,
        ],
        compiler_params=pltpu.CompilerParams(needs_layout_passes=False, use_tc_tiling_on_sc=False),
    )
    def agg(fp_hbm, s_hbm, invd_hbm, src_hbm, dst_hbm, z_hbm, out_hbm,
            isrc, idst, sbuf, ibuf, abuf, fpbuf, ish, rsh, sem1, sem2):
        cid = lax.axis_index("c")
        sid = lax.axis_index("s")
        wid = cid * NS + sid

        def stage_rows(off, size):
            pltpu.sync_copy(invd_hbm.at[pl.ds(off, size)],
                            ish.at[pl.ds(off, size)])
            pltpu.sync_copy(z_hbm.at[pl.ds(off, size)],
                            rsh.at[pl.ds(off, size)])

        _per_subcore_rows(sid, N, stage_rows)
        plsc.subcore_barrier()
        iota = lax.iota(jnp.int32, L)
        cols = [jnp.full((L,), h, jnp.int32) for h in range(H)]

        def chunk(k, carry):
            base = wid * EPW + k * C
            pltpu.sync_copy(src_hbm.at[pl.ds(base, C)], isrc)
            pltpu.sync_copy(dst_hbm.at[pl.ds(base, C)], idst)
            pltpu.sync_copy(s_hbm.at[pl.ds(base, C)], sbuf)
            d1 = pltpu.make_async_copy(fp_hbm.at[isrc], fpbuf, sem1)
            d2 = pltpu.make_async_copy(ish.at[idst], ibuf, sem2)
            d1.start()
            d2.start()
            d2.wait()
            d1.wait()
            for g in range((C * H) // L):
                p = g * L + iota
                r = p >> 3
                c = p & 7
                av = plsc.load_gather(sbuf, [r, c]) * plsc.load_gather(ibuf, [r, c])
                plsc.store_scatter(abuf, [r, c], av)
            def edge(e, carry2):
                eb = jnp.full((L,), e, jnp.int32)
                for h in range(H):
                    a = plsc.load_gather(abuf, [eb, cols[h]])
                    v = fpbuf[e, pl.ds(h * L, L)]
                    fpbuf[e, pl.ds(h * L, L)] = v * a
                return carry2

            lax.fori_loop(0, C, edge, 0)
            pltpu.sync_copy(fpbuf, rsh.at[idst], add=True)
            return carry

        lax.fori_loop(0, NCH, chunk, 0)
        plsc.subcore_barrier()

        def export_rows(off, size):
            pltpu.sync_copy(rsh.at[pl.ds(off, size)],
                            out_hbm.at[cid, pl.ds(off, size)])

        _per_subcore_rows(sid, N, export_rows)

    return agg


@jax.jit
def kernel(edge_index, feat, W, attn_l, attn_r, bias):
    N, F = feat.shape
    HD = W.shape[0]
    H = attn_l.shape[1]
    D = attn_l.shape[2]
    E = edge_index.shape[1]
    src = edge_index[0]
    dst = edge_index[1]

    wl = (W.reshape(H, D, F) * attn_l.reshape(H, D, 1)).sum(axis=1)
    wr = (W.reshape(H, D, F) * attn_r.reshape(H, D, 1)).sum(axis=1)
    wcat = jnp.concatenate([wl, wr], axis=0)

    RB = 1000
    fp, elr = pl.pallas_call(
        _proj_body,
        grid=(N // RB,),
        in_specs=[
            pl.BlockSpec((RB, F), lambda i: (i, 0)),
            pl.BlockSpec((HD, F), lambda i: (0, 0)),
            pl.BlockSpec((2 * H, F), lambda i: (0, 0)),
        ],
        out_specs=[
            pl.BlockSpec((RB, HD), lambda i: (i, 0)),
            pl.BlockSpec((RB, 2 * H), lambda i: (i, 0)),
        ],
        out_shape=[
            jax.ShapeDtypeStruct((N, HD), jnp.float32),
            jax.ShapeDtypeStruct((N, 2 * H), jnp.float32),
        ],
    )(feat, W, wcat)

    zeros_h = jnp.zeros((N, H), jnp.float32)
    s, dparts = _make_scores(N, E, H)(elr, src, dst, zeros_h)

    invd = pl.pallas_call(
        _invd_body,
        out_shape=jax.ShapeDtypeStruct((N * H // 128, 128), jnp.float32),
    )(dparts.reshape(NC, N * H // 128, 128))
    invd = invd.reshape(N, H)

    zeros_hd = jnp.zeros((N, HD), jnp.float32)
    rparts = _make_agg(N, E, H, HD)(fp, s, invd, src, dst, zeros_hd)

    rst = pl.pallas_call(
        _merge_body,
        grid=(N // RB,),
        in_specs=[
            pl.BlockSpec((NC, RB, HD), lambda i: (0, i, 0)),
            pl.BlockSpec((1, HD), lambda i: (0, 0)),
        ],
        out_specs=pl.BlockSpec((RB, HD), lambda i: (i, 0)),
        out_shape=jax.ShapeDtypeStruct((N, HD), jnp.float32),
    )(rparts, bias.reshape(1, HD))

    return rst.reshape(N, H, D)


# speedup vs baseline: 43.8305x; 1.2808x over previous
import functools

import jax
import jax.numpy as jnp
from jax import lax
from jax.experimental import pallas as pl
from jax.experimental.pallas import tpu as pltpu
from jax.experimental.pallas import tpu_sc as plsc

NEG_SLOPE = 0.2
NC = 2
NS = 16
NW = NC * NS
L = 16


def _per_subcore_rows(sid, n_rows, copy_fn):
    rpb = (n_rows // NS) & ~7
    last = n_rows - (NS - 1) * rpb

    @pl.when(sid < NS - 1)
    def _():
        copy_fn(pl.multiple_of(sid * rpb, 8), rpb)

    @pl.when(sid == NS - 1)
    def _():
        copy_fn((NS - 1) * rpb, last)


def _proj_body(x_ref, w_ref, wc_ref, fp_ref, elr_ref):
    x = x_ref[...]
    fp_ref[...] = lax.dot_general(
        x, w_ref[...], (((1,), (1,)), ((), ())),
        preferred_element_type=jnp.float32)
    elr_ref[...] = lax.dot_general(
        x, wc_ref[...], (((1,), (1,)), ((), ())),
        preferred_element_type=jnp.float32)


def _invd_body(d_ref, o_ref):
    d = d_ref[...]
    o_ref[...] = 1.0 / (d[0] + d[1] + 1e-9)


def _merge_body(r_ref, b_ref, o_ref):
    r = r_ref[...]
    o_ref[...] = r[0] + r[1] + b_ref[...]


def _make_scores(N, E, H):
    C = 128
    NCHG = E // C
    mesh = plsc.VectorSubcoreMesh(core_axis_name="c", subcore_axis_name="s")

    @functools.partial(
        pl.kernel,
        out_type=[
            jax.ShapeDtypeStruct((E, H), jnp.float32),
            jax.ShapeDtypeStruct((NC, N, H), jnp.float32),
        ],
        mesh=mesh,
        scratch_types=[
            pltpu.VMEM((2, C), jnp.int32),
            pltpu.VMEM((C, 2 * H), jnp.float32),
            pltpu.VMEM((C, 2 * H), jnp.float32),
            pltpu.VMEM((C, H), jnp.float32),
            pltpu.VMEM_SHARED((N, H), jnp.float32),
            pltpu.SemaphoreType.DMA,
            pltpu.SemaphoreType.DMA,
        ],
        compiler_params=pltpu.CompilerParams(needs_layout_passes=False, use_tc_tiling_on_sc=False),
    )
    def scores(edge_hbm, elr_hbm, z_hbm, s_hbm, dpart_hbm,
               iedge, rsrc, rdst, sbuf, dsh, sem1, sem2):
        cid = lax.axis_index("c")
        sid = lax.axis_index("s")
        wid = cid * NS + sid
        nch_w = NCHG // NW + jnp.where(wid < NCHG % NW, 1, 0)

        def zero_rows(off, size):
            pltpu.sync_copy(z_hbm.at[pl.ds(off, size)],
                            dsh.at[pl.ds(off, size)])

        _per_subcore_rows(sid, N, zero_rows)
        plsc.subcore_barrier()
        iota = lax.iota(jnp.int32, L)
        cols_l = [jnp.full((L,), h, jnp.int32) for h in range(H)]
        cols_r = [jnp.full((L,), H + h, jnp.int32) for h in range(H)]

        def chunk(k, carry):
            base = (wid + k * NW) * C
            pltpu.sync_copy(edge_hbm.at[:, pl.ds(base, C)], iedge)
            d1 = pltpu.async_copy(elr_hbm.at[iedge.at[0]], rsrc, sem1)
            d2 = pltpu.async_copy(elr_hbm.at[iedge.at[1]], rdst, sem2)
            d1.wait()
            d2.wait()
            for g in range(C // L):
                rows = g * L + iota
                for h in range(H):
                    el = plsc.load_gather(rsrc, [rows, cols_l[h]])
                    er = plsc.load_gather(rdst, [rows, cols_r[h]])
                    e = el + er
                    e = jnp.where(e > 0.0, e, NEG_SLOPE * e)
                    plsc.store_scatter(sbuf, [rows, cols_l[h]], jnp.exp(e))
            pltpu.sync_copy(sbuf, s_hbm.at[pl.ds(base, C)])
            pltpu.sync_copy(sbuf, dsh.at[iedge.at[1]], add=True)
            return carry

        lax.fori_loop(0, nch_w, chunk, 0)
        plsc.subcore_barrier()

        def export_rows(off, size):
            pltpu.sync_copy(dsh.at[pl.ds(off, size)],
                            dpart_hbm.at[cid, pl.ds(off, size)])

        _per_subcore_rows(sid, N, export_rows)

    return scores


def _make_agg(N, E, H, HD):
    C = 128
    NCHG = E // C
    mesh = plsc.VectorSubcoreMesh(core_axis_name="c", subcore_axis_name="s")

    @functools.partial(
        pl.kernel,
        out_type=jax.ShapeDtypeStruct((NC, N, HD), jnp.float32),
        mesh=mesh,
        scratch_types=[
            pltpu.VMEM((2, C), jnp.int32),
            pltpu.VMEM((C, H), jnp.float32),
            pltpu.VMEM((C, H), jnp.float32),
            pltpu.VMEM((C, H), jnp.float32),
            pltpu.VMEM((C, HD), jnp.float32),
            pltpu.VMEM_SHARED((N, H), jnp.float32),
            pltpu.VMEM_SHARED((N, HD), jnp.float32),
            pltpu.SemaphoreType.DMA,
            pltpu.SemaphoreType.DMA,
        ],
        compiler_params=pltpu.CompilerParams(needs_layout_passes=False, use_tc_tiling_on_sc=False),
    )
    def agg(edge_hbm, fp_hbm, s_hbm, invd_hbm, z_hbm, out_hbm,
            iedge, sbuf, ibuf, abuf, fpbuf, ish, rsh, sem1, sem2):
        cid = lax.axis_index("c")
        sid = lax.axis_index("s")
        wid = cid * NS + sid
        nch_w = NCHG // NW + jnp.where(wid < NCHG % NW, 1, 0)

        def stage_rows(off, size):
            pltpu.sync_copy(invd_hbm.at[pl.ds(off, size)],
                            ish.at[pl.ds(off, size)])
            pltpu.sync_copy(z_hbm.at[pl.ds(off, size)],
                            rsh.at[pl.ds(off, size)])

        _per_subcore_rows(sid, N, stage_rows)
        plsc.subcore_barrier()
        iota = lax.iota(jnp.int32, L)
        cols = [jnp.full((L,), h, jnp.int32) for h in range(H)]

        def chunk(k, carry):
            base = (wid + k * NW) * C
            d0 = pltpu.async_copy(edge_hbm.at[:, pl.ds(base, C)], iedge,
                                  sem1)
            d1 = pltpu.async_copy(s_hbm.at[pl.ds(base, C)], sbuf, sem2)
            d0.wait()
            d1.wait()
            d2 = pltpu.async_copy(fp_hbm.at[iedge.at[0]], fpbuf, sem1)
            d3 = pltpu.async_copy(ish.at[iedge.at[1]], ibuf, sem2)
            d3.wait()
            d2.wait()
            for g in range((C * H) // L):
                p = g * L + iota
                r = p >> 3
                c = p & 7
                av = (plsc.load_gather(sbuf, [r, c])
                      * plsc.load_gather(ibuf, [r, c]))
                plsc.store_scatter(abuf, [r, c], av)

            def epair(e2, carry2):
                for q in range(2):
                    e = e2 * 2 + q
                    eb = jnp.full((L,), e, jnp.int32)
                    for h in range(H):
                        a = plsc.load_gather(abuf, [eb, cols[h]])
                        v = fpbuf[e, pl.ds(h * L, L)]
                        fpbuf[e, pl.ds(h * L, L)] = v * a
                return carry2

            lax.fori_loop(0, C // 2, epair, 0)
            pltpu.sync_copy(fpbuf, rsh.at[iedge.at[1]], add=True)
            return carry

        lax.fori_loop(0, nch_w, chunk, 0)
        plsc.subcore_barrier()

        def export_rows(off, size):
            pltpu.sync_copy(rsh.at[pl.ds(off, size)],
                            out_hbm.at[cid, pl.ds(off, size)])

        _per_subcore_rows(sid, N, export_rows)

    return agg


@jax.jit
def kernel(edge_index, feat, W, attn_l, attn_r, bias):
    N, F = feat.shape
    HD = W.shape[0]
    H = attn_l.shape[1]
    D = attn_l.shape[2]
    E = edge_index.shape[1]

    wl = (W.reshape(H, D, F) * attn_l.reshape(H, D, 1)).sum(axis=1)
    wr = (W.reshape(H, D, F) * attn_r.reshape(H, D, 1)).sum(axis=1)
    wcat = jnp.concatenate([wl, wr], axis=0)

    RB = 1000
    fp, elr = pl.pallas_call(
        _proj_body,
        grid=(N // RB,),
        in_specs=[
            pl.BlockSpec((RB, F), lambda i: (i, 0)),
            pl.BlockSpec((HD, F), lambda i: (0, 0)),
            pl.BlockSpec((2 * H, F), lambda i: (0, 0)),
        ],
        out_specs=[
            pl.BlockSpec((RB, HD), lambda i: (i, 0)),
            pl.BlockSpec((RB, 2 * H), lambda i: (i, 0)),
        ],
        out_shape=[
            jax.ShapeDtypeStruct((N, HD), jnp.float32),
            jax.ShapeDtypeStruct((N, 2 * H), jnp.float32),
        ],
    )(feat, W, wcat)

    zeros_h = jnp.zeros((N, H), jnp.float32)
    s, dparts = _make_scores(N, E, H)(edge_index, elr, zeros_h)

    invd = pl.pallas_call(
        _invd_body,
        out_shape=jax.ShapeDtypeStruct((N * H // 128, 128), jnp.float32),
    )(dparts.reshape(NC, N * H // 128, 128))
    invd = invd.reshape(N, H)

    zeros_hd = jnp.zeros((N, HD), jnp.float32)
    rparts = _make_agg(N, E, H, HD)(edge_index, fp, s, invd, zeros_hd)

    rst = pl.pallas_call(
        _merge_body,
        grid=(N // RB,),
        in_specs=[
            pl.BlockSpec((NC, RB, HD), lambda i: (0, i, 0)),
            pl.BlockSpec((1, HD), lambda i: (0, 0)),
        ],
        out_specs=pl.BlockSpec((RB, HD), lambda i: (i, 0)),
        out_shape=jax.ShapeDtypeStruct((N, HD), jnp.float32),
    )(rparts, bias.reshape(1, HD))

    return rst.reshape(N, H, D)


# speedup vs baseline: 47.1857x; 1.0766x over previous
import functools

import jax
import jax.numpy as jnp
from jax import lax
from jax.experimental import pallas as pl
from jax.experimental.pallas import tpu as pltpu
from jax.experimental.pallas import tpu_sc as plsc

NEG_SLOPE = 0.2
NC = 2
NS = 16
NW = NC * NS
L = 16


def _per_subcore_rows(sid, n_rows, copy_fn):
    rpb = (n_rows // NS) & ~7
    last = n_rows - (NS - 1) * rpb

    @pl.when(sid < NS - 1)
    def _():
        copy_fn(pl.multiple_of(sid * rpb, 8), rpb)

    @pl.when(sid == NS - 1)
    def _():
        copy_fn((NS - 1) * rpb, last)


def _proj_body(x_ref, w_ref, wc_ref, fp_ref, elr_ref):
    x = x_ref[...]
    fp_ref[...] = lax.dot_general(
        x, w_ref[...], (((1,), (1,)), ((), ())),
        preferred_element_type=jnp.float32)
    elr_ref[...] = lax.dot_general(
        x, wc_ref[...], (((1,), (1,)), ((), ())),
        preferred_element_type=jnp.float32)


def _invd_body(d_ref, o_ref):
    d = d_ref[...]
    o_ref[...] = 1.0 / (d[0] + d[1] + 1e-9)


def _merge_body(r_ref, b_ref, o_ref):
    r = r_ref[...]
    o_ref[...] = r[0] + r[1] + b_ref[...]


def _make_scores(N, E, H):
    C2 = 128
    C = 2 * C2
    NCHG = E // C
    mesh = plsc.VectorSubcoreMesh(core_axis_name="c", subcore_axis_name="s")

    @functools.partial(
        pl.kernel,
        out_type=[
            jax.ShapeDtypeStruct((E, H), jnp.float32),
            jax.ShapeDtypeStruct((NC, N, H), jnp.float32),
        ],
        mesh=mesh,
        scratch_types=[
            pltpu.VMEM((2, 2, C2), jnp.int32),
            pltpu.VMEM((2, C2, 2 * H), jnp.float32),
            pltpu.VMEM((2, C2, 2 * H), jnp.float32),
            pltpu.VMEM((2, C2, H), jnp.float32),
            pltpu.VMEM_SHARED((N, H), jnp.float32),
            pltpu.SemaphoreType.DMA,
            pltpu.SemaphoreType.DMA,
            pltpu.SemaphoreType.DMA,
            pltpu.SemaphoreType.DMA,
        ],
        compiler_params=pltpu.CompilerParams(needs_layout_passes=False, use_tc_tiling_on_sc=False),
    )
    def scores(edge_hbm, elr_hbm, z_hbm, s_hbm, dpart_hbm,
               iedge, rsrc, rdst, sbuf, dsh, sem1, sem2, sem3, sem4):
        cid = lax.axis_index("c")
        sid = lax.axis_index("s")
        wid = cid * NS + sid
        nch_w = NCHG // NW + jnp.where(wid < NCHG % NW, 1, 0)

        def zero_rows(off, size):
            pltpu.sync_copy(z_hbm.at[pl.ds(off, size)],
                            dsh.at[pl.ds(off, size)])

        _per_subcore_rows(sid, N, zero_rows)
        plsc.subcore_barrier()
        iota = lax.iota(jnp.int32, L)
        cols_l = [jnp.full((L,), h, jnp.int32) for h in range(H)]
        cols_r = [jnp.full((L,), H + h, jnp.int32) for h in range(H)]

        def half_compute(base, h2):
            for g in range(C2 // L):
                rows = g * L + iota
                for h in range(H):
                    el = plsc.load_gather(rsrc.at[h2], [rows, cols_l[h]])
                    er = plsc.load_gather(rdst.at[h2], [rows, cols_r[h]])
                    e = el + er
                    e = jnp.where(e > 0.0, e, NEG_SLOPE * e)
                    plsc.store_scatter(sbuf.at[h2], [rows, cols_l[h]],
                                       jnp.exp(e))
            pltpu.sync_copy(sbuf.at[h2],
                            s_hbm.at[pl.ds(base + h2 * C2, C2)])
            pltpu.sync_copy(sbuf.at[h2], dsh.at[iedge.at[h2, 1]], add=True)

        def chunk(k, carry):
            base = (wid + k * NW) * C
            e0 = pltpu.async_copy(
                edge_hbm.at[:, pl.ds(base, C2)], iedge.at[0], sem1)
            e1 = pltpu.async_copy(
                edge_hbm.at[:, pl.ds(base + C2, C2)], iedge.at[1], sem2)
            e0.wait()
            e1.wait()
            g0s = pltpu.async_copy(elr_hbm.at[iedge.at[0, 0]], rsrc.at[0],
                                   sem1)
            g0d = pltpu.async_copy(elr_hbm.at[iedge.at[0, 1]], rdst.at[0],
                                   sem3)
            g1s = pltpu.async_copy(elr_hbm.at[iedge.at[1, 0]], rsrc.at[1],
                                   sem2)
            g1d = pltpu.async_copy(elr_hbm.at[iedge.at[1, 1]], rdst.at[1],
                                   sem4)
            g0s.wait()
            g0d.wait()
            half_compute(base, 0)
            g1s.wait()
            g1d.wait()
            half_compute(base, 1)
            return carry

        lax.fori_loop(0, nch_w, chunk, 0)
        plsc.subcore_barrier()

        def export_rows(off, size):
            pltpu.sync_copy(dsh.at[pl.ds(off, size)],
                            dpart_hbm.at[cid, pl.ds(off, size)])

        _per_subcore_rows(sid, N, export_rows)

    return scores


def _make_agg(N, E, H, HD):
    C2 = 128
    C = 2 * C2
    NCHG = E // C
    mesh = plsc.VectorSubcoreMesh(core_axis_name="c", subcore_axis_name="s")

    @functools.partial(
        pl.kernel,
        out_type=jax.ShapeDtypeStruct((NC, N, HD), jnp.float32),
        mesh=mesh,
        scratch_types=[
            pltpu.VMEM((2, 2, C2), jnp.int32),
            pltpu.VMEM((2, C2, H), jnp.float32),
            pltpu.VMEM((2, C2, H), jnp.float32),
            pltpu.VMEM((2, C2, H), jnp.float32),
            pltpu.VMEM((2, C2, HD), jnp.float32),
            pltpu.VMEM_SHARED((N, H), jnp.float32),
            pltpu.VMEM_SHARED((N, HD), jnp.float32),
            pltpu.SemaphoreType.DMA,
            pltpu.SemaphoreType.DMA,
            pltpu.SemaphoreType.DMA,
            pltpu.SemaphoreType.DMA,
        ],
        compiler_params=pltpu.CompilerParams(needs_layout_passes=False, use_tc_tiling_on_sc=False),
    )
    def agg(edge_hbm, fp_hbm, s_hbm, invd_hbm, z_hbm, out_hbm,
            iedge, sbuf, ibuf, abuf, fpbuf, ish, rsh,
            sem1, sem2, sem3, sem4):
        cid = lax.axis_index("c")
        sid = lax.axis_index("s")
        wid = cid * NS + sid
        nch_w = NCHG // NW + jnp.where(wid < NCHG % NW, 1, 0)

        def stage_rows(off, size):
            pltpu.sync_copy(invd_hbm.at[pl.ds(off, size)],
                            ish.at[pl.ds(off, size)])
            pltpu.sync_copy(z_hbm.at[pl.ds(off, size)],
                            rsh.at[pl.ds(off, size)])

        _per_subcore_rows(sid, N, stage_rows)
        plsc.subcore_barrier()
        iota = lax.iota(jnp.int32, L)
        cols = [jnp.full((L,), h, jnp.int32) for h in range(H)]

        def half_compute(h2):
            for g in range((C2 * H) // L):
                p = g * L + iota
                r = p >> 3
                c = p & 7
                av = (plsc.load_gather(sbuf.at[h2], [r, c])
                      * plsc.load_gather(ibuf.at[h2], [r, c]))
                plsc.store_scatter(abuf.at[h2], [r, c], av)

            def epair(e2, carry2):
                for q in range(2):
                    e = e2 * 2 + q
                    eb = jnp.full((L,), e, jnp.int32)
                    for h in range(H):
                        a = plsc.load_gather(abuf.at[h2], [eb, cols[h]])
                        v = fpbuf[h2, e, pl.ds(h * L, L)]
                        fpbuf[h2, e, pl.ds(h * L, L)] = v * a
                return carry2

            lax.fori_loop(0, C2 // 2, epair, 0)
            pltpu.sync_copy(fpbuf.at[h2], rsh.at[iedge.at[h2, 1]],
                            add=True)

        def chunk(k, carry):
            base = (wid + k * NW) * C
            e0 = pltpu.async_copy(
                edge_hbm.at[:, pl.ds(base, C2)], iedge.at[0], sem1)
            e1 = pltpu.async_copy(
                edge_hbm.at[:, pl.ds(base + C2, C2)], iedge.at[1], sem2)
            s0 = pltpu.async_copy(
                s_hbm.at[pl.ds(base, C2)], sbuf.at[0], sem3)
            s1 = pltpu.async_copy(
                s_hbm.at[pl.ds(base + C2, C2)], sbuf.at[1], sem4)
            e0.wait()
            e1.wait()
            s0.wait()
            s1.wait()
            f0 = pltpu.async_copy(fp_hbm.at[iedge.at[0, 0]], fpbuf.at[0],
                                  sem1)
            i0 = pltpu.async_copy(ish.at[iedge.at[0, 1]], ibuf.at[0],
                                  sem3)
            f1 = pltpu.async_copy(fp_hbm.at[iedge.at[1, 0]], fpbuf.at[1],
                                  sem2)
            i1 = pltpu.async_copy(ish.at[iedge.at[1, 1]], ibuf.at[1],
                                  sem4)
            i0.wait()
            f0.wait()
            half_compute(0)
            i1.wait()
            f1.wait()
            half_compute(1)
            return carry

        lax.fori_loop(0, nch_w, chunk, 0)
        plsc.subcore_barrier()

        def export_rows(off, size):
            pltpu.sync_copy(rsh.at[pl.ds(off, size)],
                            out_hbm.at[cid, pl.ds(off, size)])

        _per_subcore_rows(sid, N, export_rows)

    return agg


@jax.jit
def kernel(edge_index, feat, W, attn_l, attn_r, bias):
    N, F = feat.shape
    HD = W.shape[0]
    H = attn_l.shape[1]
    D = attn_l.shape[2]
    E = edge_index.shape[1]

    wl = (W.reshape(H, D, F) * attn_l.reshape(H, D, 1)).sum(axis=1)
    wr = (W.reshape(H, D, F) * attn_r.reshape(H, D, 1)).sum(axis=1)
    wcat = jnp.concatenate([wl, wr], axis=0)

    RB = 1000
    fp, elr = pl.pallas_call(
        _proj_body,
        grid=(N // RB,),
        in_specs=[
            pl.BlockSpec((RB, F), lambda i: (i, 0)),
            pl.BlockSpec((HD, F), lambda i: (0, 0)),
            pl.BlockSpec((2 * H, F), lambda i: (0, 0)),
        ],
        out_specs=[
            pl.BlockSpec((RB, HD), lambda i: (i, 0)),
            pl.BlockSpec((RB, 2 * H), lambda i: (i, 0)),
        ],
        out_shape=[
            jax.ShapeDtypeStruct((N, HD), jnp.float32),
            jax.ShapeDtypeStruct((N, 2 * H), jnp.float32),
        ],
    )(feat, W, wcat)

    zeros_h = jnp.zeros((N, H), jnp.float32)
    s, dparts = _make_scores(N, E, H)(edge_index, elr, zeros_h)

    invd = pl.pallas_call(
        _invd_body,
        out_shape=jax.ShapeDtypeStruct((N * H // 128, 128), jnp.float32),
    )(dparts.reshape(NC, N * H // 128, 128))
    invd = invd.reshape(N, H)

    zeros_hd = jnp.zeros((N, HD), jnp.float32)
    rparts = _make_agg(N, E, H, HD)(edge_index, fp, s, invd, zeros_hd)

    rst = pl.pallas_call(
        _merge_body,
        grid=(N // RB,),
        in_specs=[
            pl.BlockSpec((NC, RB, HD), lambda i: (0, i, 0)),
            pl.BlockSpec((1, HD), lambda i: (0, 0)),
        ],
        out_specs=pl.BlockSpec((RB, HD), lambda i: (i, 0)),
        out_shape=jax.ShapeDtypeStruct((N, HD), jnp.float32),
    )(rparts, bias.reshape(1, HD))

    return rst.reshape(N, H, D)


# speedup vs baseline: 47.2346x; 1.0010x over previous
import functools

import jax
import jax.numpy as jnp
from jax import lax
from jax.experimental import pallas as pl
from jax.experimental.pallas import tpu as pltpu
from jax.experimental.pallas import tpu_sc as plsc

NEG_SLOPE = 0.2
NC = 2
NS = 16
NW = NC * NS
L = 16


def _per_subcore_rows(sid, n_rows, copy_fn):
    rpb = (n_rows // NS) & ~7
    last = n_rows - (NS - 1) * rpb

    @pl.when(sid < NS - 1)
    def _():
        copy_fn(pl.multiple_of(sid * rpb, 8), rpb)

    @pl.when(sid == NS - 1)
    def _():
        copy_fn((NS - 1) * rpb, last)


def _proj_body(x_ref, w_ref, wc_ref, fp_ref, elr_ref):
    x = x_ref[...]
    fp_ref[...] = lax.dot_general(
        x, w_ref[...], (((1,), (1,)), ((), ())),
        preferred_element_type=jnp.float32)
    elr_ref[...] = lax.dot_general(
        x, wc_ref[...], (((1,), (1,)), ((), ())),
        preferred_element_type=jnp.float32)


def _invd_body(d_ref, o_ref):
    d = d_ref[...]
    o_ref[...] = 1.0 / (d[0] + d[1] + 1e-9)


def _merge_body(r_ref, b_ref, o_ref):
    r = r_ref[...]
    o_ref[...] = r[0] + r[1] + b_ref[...]


def _make_scores(N, E, H):
    C2 = 128
    C = 2 * C2
    NCHG = E // C
    mesh = plsc.VectorSubcoreMesh(core_axis_name="c", subcore_axis_name="s")

    @functools.partial(
        pl.kernel,
        out_type=[
            jax.ShapeDtypeStruct((E, H), jnp.float32),
            jax.ShapeDtypeStruct((NC, N, H), jnp.float32),
        ],
        mesh=mesh,
        scratch_types=[
            pltpu.VMEM((2, 2, C2), jnp.int32),
            pltpu.VMEM((2, C2, 2 * H), jnp.float32),
            pltpu.VMEM((2, C2, 2 * H), jnp.float32),
            pltpu.VMEM((2, C2, H), jnp.float32),
            pltpu.VMEM_SHARED((N, H), jnp.float32),
            pltpu.SemaphoreType.DMA,
            pltpu.SemaphoreType.DMA,
            pltpu.SemaphoreType.DMA,
            pltpu.SemaphoreType.DMA,
        ],
        compiler_params=pltpu.CompilerParams(needs_layout_passes=False, use_tc_tiling_on_sc=False),
    )
    def scores(edge_hbm, elr_hbm, z_hbm, s_hbm, dpart_hbm,
               iedge, rsrc, rdst, sbuf, dsh, sem1, sem2, sem3, sem4):
        cid = lax.axis_index("c")
        sid = lax.axis_index("s")
        wid = cid * NS + sid
        nch_w = NCHG // NW + jnp.where(wid < NCHG % NW, 1, 0)

        def zero_rows(off, size):
            pltpu.sync_copy(z_hbm.at[pl.ds(off, size)],
                            dsh.at[pl.ds(off, size)])

        _per_subcore_rows(sid, N, zero_rows)
        plsc.subcore_barrier()
        iota = lax.iota(jnp.int32, L)
        cols_l = [jnp.full((L,), h, jnp.int32) for h in range(H)]
        cols_r = [jnp.full((L,), H + h, jnp.int32) for h in range(H)]

        def half_compute(base, h2):
            for g in range(C2 // L):
                rows = g * L + iota
                for h in range(H):
                    el = plsc.load_gather(rsrc.at[h2], [rows, cols_l[h]])
                    er = plsc.load_gather(rdst.at[h2], [rows, cols_r[h]])
                    e = el + er
                    e = jnp.where(e > 0.0, e, NEG_SLOPE * e)
                    plsc.store_scatter(sbuf.at[h2], [rows, cols_l[h]],
                                       jnp.exp(e))
            pltpu.sync_copy(sbuf.at[h2],
                            s_hbm.at[pl.ds(base + h2 * C2, C2)])
            pltpu.sync_copy(sbuf.at[h2], dsh.at[iedge.at[h2, 1]], add=True)

        def chunk(k, carry):
            base = (wid + k * NW) * C
            e0 = pltpu.async_copy(
                edge_hbm.at[:, pl.ds(base, C2)], iedge.at[0], sem1)
            e1 = pltpu.async_copy(
                edge_hbm.at[:, pl.ds(base + C2, C2)], iedge.at[1], sem2)
            e0.wait()
            e1.wait()
            g0s = pltpu.async_copy(elr_hbm.at[iedge.at[0, 0]], rsrc.at[0],
                                   sem1)
            g0d = pltpu.async_copy(elr_hbm.at[iedge.at[0, 1]], rdst.at[0],
                                   sem3)
            g1s = pltpu.async_copy(elr_hbm.at[iedge.at[1, 0]], rsrc.at[1],
                                   sem2)
            g1d = pltpu.async_copy(elr_hbm.at[iedge.at[1, 1]], rdst.at[1],
                                   sem4)
            g0s.wait()
            g0d.wait()
            half_compute(base, 0)
            g1s.wait()
            g1d.wait()
            half_compute(base, 1)
            return carry

        lax.fori_loop(0, nch_w, chunk, 0)
        plsc.subcore_barrier()

        def export_rows(off, size):
            pltpu.sync_copy(dsh.at[pl.ds(off, size)],
                            dpart_hbm.at[cid, pl.ds(off, size)])

        _per_subcore_rows(sid, N, export_rows)

    return scores


def _make_agg(N, E, H, HD):
    C2 = 128
    C = 2 * C2
    NCHG = E // C
    mesh = plsc.VectorSubcoreMesh(core_axis_name="c", subcore_axis_name="s")

    @functools.partial(
        pl.kernel,
        out_type=jax.ShapeDtypeStruct((NC, N, HD), jnp.float32),
        mesh=mesh,
        scratch_types=[
            pltpu.VMEM((2, 2, C2), jnp.int32),
            pltpu.VMEM((2, C2, H), jnp.float32),
            pltpu.VMEM((2, C2, H), jnp.float32),
            pltpu.VMEM((2, C2, H), jnp.float32),
            pltpu.VMEM((2, C2, HD), jnp.float32),
            pltpu.VMEM_SHARED((N, H), jnp.float32),
            pltpu.VMEM_SHARED((N, HD), jnp.float32),
            pltpu.SemaphoreType.DMA,
            pltpu.SemaphoreType.DMA,
            pltpu.SemaphoreType.DMA,
            pltpu.SemaphoreType.DMA,
        ],
        compiler_params=pltpu.CompilerParams(needs_layout_passes=False, use_tc_tiling_on_sc=False),
    )
    def agg(edge_hbm, fp_hbm, s_hbm, invd_hbm, z_hbm, out_hbm,
            iedge, sbuf, ibuf, abuf, fpbuf, ish, rsh,
            sem1, sem2, sem3, sem4):
        cid = lax.axis_index("c")
        sid = lax.axis_index("s")
        wid = cid * NS + sid
        nch_w = NCHG // NW + jnp.where(wid < NCHG % NW, 1, 0)

        def stage_rows(off, size):
            pltpu.sync_copy(invd_hbm.at[pl.ds(off, size)],
                            ish.at[pl.ds(off, size)])
            pltpu.sync_copy(z_hbm.at[pl.ds(off, size)],
                            rsh.at[pl.ds(off, size)])

        _per_subcore_rows(sid, N, stage_rows)
        plsc.subcore_barrier()
        iota = lax.iota(jnp.int32, L)
        cols = [jnp.full((L,), h, jnp.int32) for h in range(H)]

        def half_compute(h2):
            for g in range((C2 * H) // L):
                p = g * L + iota
                r = p >> 3
                c = p & 7
                av = (plsc.load_gather(sbuf.at[h2], [r, c])
                      * plsc.load_gather(ibuf.at[h2], [r, c]))
                plsc.store_scatter(abuf.at[h2], [r, c], av)

            def equad(e4, carry2):
                for q in range(4):
                    e = e4 * 4 + q
                    eb = jnp.full((L,), e, jnp.int32)
                    for h in range(H):
                        a = plsc.load_gather(abuf.at[h2], [eb, cols[h]])
                        v = fpbuf[h2, e, pl.ds(h * L, L)]
                        fpbuf[h2, e, pl.ds(h * L, L)] = v * a
                return carry2

            lax.fori_loop(0, C2 // 4, equad, 0)
            pltpu.sync_copy(fpbuf.at[h2], rsh.at[iedge.at[h2, 1]],
                            add=True)

        def chunk(k, carry):
            base = (wid + k * NW) * C
            e0 = pltpu.async_copy(
                edge_hbm.at[:, pl.ds(base, C2)], iedge.at[0], sem1)
            e1 = pltpu.async_copy(
                edge_hbm.at[:, pl.ds(base + C2, C2)], iedge.at[1], sem2)
            s0 = pltpu.async_copy(
                s_hbm.at[pl.ds(base, C2)], sbuf.at[0], sem3)
            s1 = pltpu.async_copy(
                s_hbm.at[pl.ds(base + C2, C2)], sbuf.at[1], sem4)
            e0.wait()
            e1.wait()
            s0.wait()
            s1.wait()
            f0 = pltpu.async_copy(fp_hbm.at[iedge.at[0, 0]], fpbuf.at[0],
                                  sem1)
            i0 = pltpu.async_copy(ish.at[iedge.at[0, 1]], ibuf.at[0],
                                  sem3)
            f1 = pltpu.async_copy(fp_hbm.at[iedge.at[1, 0]], fpbuf.at[1],
                                  sem2)
            i1 = pltpu.async_copy(ish.at[iedge.at[1, 1]], ibuf.at[1],
                                  sem4)
            i0.wait()
            f0.wait()
            half_compute(0)
            i1.wait()
            f1.wait()
            half_compute(1)
            return carry

        lax.fori_loop(0, nch_w, chunk, 0)
        plsc.subcore_barrier()

        def export_rows(off, size):
            pltpu.sync_copy(rsh.at[pl.ds(off, size)],
                            out_hbm.at[cid, pl.ds(off, size)])

        _per_subcore_rows(sid, N, export_rows)

    return agg


@jax.jit
def kernel(edge_index, feat, W, attn_l, attn_r, bias):
    N, F = feat.shape
    HD = W.shape[0]
    H = attn_l.shape[1]
    D = attn_l.shape[2]
    E = edge_index.shape[1]

    wl = (W.reshape(H, D, F) * attn_l.reshape(H, D, 1)).sum(axis=1)
    wr = (W.reshape(H, D, F) * attn_r.reshape(H, D, 1)).sum(axis=1)
    wcat = jnp.concatenate([wl, wr], axis=0)

    RB = 1000
    fp, elr = pl.pallas_call(
        _proj_body,
        grid=(N // RB,),
        in_specs=[
            pl.BlockSpec((RB, F), lambda i: (i, 0)),
            pl.BlockSpec((HD, F), lambda i: (0, 0)),
            pl.BlockSpec((2 * H, F), lambda i: (0, 0)),
        ],
        out_specs=[
            pl.BlockSpec((RB, HD), lambda i: (i, 0)),
            pl.BlockSpec((RB, 2 * H), lambda i: (i, 0)),
        ],
        out_shape=[
            jax.ShapeDtypeStruct((N, HD), jnp.float32),
            jax.ShapeDtypeStruct((N, 2 * H), jnp.float32),
        ],
    )(feat, W, wcat)

    zeros_h = jnp.zeros((N, H), jnp.float32)
    s, dparts = _make_scores(N, E, H)(edge_index, elr, zeros_h)

    invd = pl.pallas_call(
        _invd_body,
        out_shape=jax.ShapeDtypeStruct((N * H // 128, 128), jnp.float32),
    )(dparts.reshape(NC, N * H // 128, 128))
    invd = invd.reshape(N, H)

    zeros_hd = jnp.zeros((N, HD), jnp.float32)
    rparts = _make_agg(N, E, H, HD)(edge_index, fp, s, invd, zeros_hd)

    rst = pl.pallas_call(
        _merge_body,
        grid=(N // RB,),
        in_specs=[
            pl.BlockSpec((NC, RB, HD), lambda i: (0, i, 0)),
            pl.BlockSpec((1, HD), lambda i: (0, 0)),
        ],
        out_specs=pl.BlockSpec((RB, HD), lambda i: (i, 0)),
        out_shape=jax.ShapeDtypeStruct((N, HD), jnp.float32),
    )(rparts, bias.reshape(1, HD))

    return rst.reshape(N, H, D)


# speedup vs baseline: 47.4584x; 1.0047x over previous
import functools

import jax
import jax.numpy as jnp
from jax import lax
from jax.experimental import pallas as pl
from jax.experimental.pallas import tpu as pltpu
from jax.experimental.pallas import tpu_sc as plsc

NEG_SLOPE = 0.2
NC = 2
NS = 16
NW = NC * NS
L = 16


def _per_subcore_rows(sid, n_rows, copy_fn):
    rpb = (n_rows // NS) & ~7
    last = n_rows - (NS - 1) * rpb

    @pl.when(sid < NS - 1)
    def _():
        copy_fn(pl.multiple_of(sid * rpb, 8), rpb)

    @pl.when(sid == NS - 1)
    def _():
        copy_fn((NS - 1) * rpb, last)


def _proj_body(x_ref, w_ref, wc_ref, fp_ref, elr_ref):
    x = x_ref[...]
    fp_ref[...] = lax.dot_general(
        x, w_ref[...], (((1,), (1,)), ((), ())),
        preferred_element_type=jnp.float32)
    elr_ref[...] = lax.dot_general(
        x, wc_ref[...], (((1,), (1,)), ((), ())),
        preferred_element_type=jnp.float32)


def _invd_body(d_ref, o_ref):
    d = d_ref[...]
    o_ref[...] = 1.0 / (d[0] + d[1] + 1e-9)


def _merge_body(r_ref, b_ref, o_ref):
    r = r_ref[...]
    o_ref[...] = r[0] + r[1] + b_ref[...]


def _make_scores(N, E, H):
    C2 = 128
    C = 2 * C2
    NCHG = E // C
    mesh = plsc.VectorSubcoreMesh(core_axis_name="c", subcore_axis_name="s")

    @functools.partial(
        pl.kernel,
        out_type=[
            jax.ShapeDtypeStruct((E, H), jnp.float32),
            jax.ShapeDtypeStruct((NC, N, H), jnp.float32),
        ],
        mesh=mesh,
        scratch_types=[
            pltpu.VMEM((2, 2, C2), jnp.int32),
            pltpu.VMEM((2, C2, 2 * H), jnp.float32),
            pltpu.VMEM((2, C2, 2 * H), jnp.float32),
            pltpu.VMEM((2, C2, H), jnp.float32),
            pltpu.VMEM_SHARED((N, H), jnp.float32),
            pltpu.SemaphoreType.DMA,
            pltpu.SemaphoreType.DMA,
            pltpu.SemaphoreType.DMA,
            pltpu.SemaphoreType.DMA,
            pltpu.SemaphoreType.DMA,
            pltpu.SemaphoreType.DMA,
        ],
        compiler_params=pltpu.CompilerParams(needs_layout_passes=False, use_tc_tiling_on_sc=False),
    )
    def scores(edge_hbm, elr_hbm, z_hbm, s_hbm, dpart_hbm,
               iedge, rsrc, rdst, sbuf, dsh, sem1, sem2, sem3, sem4,
               sem5, sem6):
        sems = (sem5, sem6)
        cid = lax.axis_index("c")
        sid = lax.axis_index("s")
        wid = cid * NS + sid
        nch_w = NCHG // NW + jnp.where(wid < NCHG % NW, 1, 0)

        def zero_rows(off, size):
            pltpu.sync_copy(z_hbm.at[pl.ds(off, size)],
                            dsh.at[pl.ds(off, size)])

        _per_subcore_rows(sid, N, zero_rows)
        plsc.subcore_barrier()
        iota = lax.iota(jnp.int32, L)
        cols_l = [jnp.full((L,), h, jnp.int32) for h in range(H)]
        cols_r = [jnp.full((L,), H + h, jnp.int32) for h in range(H)]

        def half_compute(base, h2):
            for g in range(C2 // L):
                rows = g * L + iota
                for h in range(H):
                    el = plsc.load_gather(rsrc.at[h2], [rows, cols_l[h]])
                    er = plsc.load_gather(rdst.at[h2], [rows, cols_r[h]])
                    e = el + er
                    e = jnp.where(e > 0.0, e, NEG_SLOPE * e)
                    plsc.store_scatter(sbuf.at[h2], [rows, cols_l[h]],
                                       jnp.exp(e))
            pltpu.async_copy(sbuf.at[h2],
                             s_hbm.at[pl.ds(base + h2 * C2, C2)],
                             sems[h2])
            pltpu.sync_copy(sbuf.at[h2], dsh.at[iedge.at[h2, 1]], add=True)

        def chunk(k, carry):
            base = (wid + k * NW) * C
            e0 = pltpu.async_copy(
                edge_hbm.at[:, pl.ds(base, C2)], iedge.at[0], sem1)
            e1 = pltpu.async_copy(
                edge_hbm.at[:, pl.ds(base + C2, C2)], iedge.at[1], sem2)
            e0.wait()
            e1.wait()
            g0s = pltpu.async_copy(elr_hbm.at[iedge.at[0, 0]], rsrc.at[0],
                                   sem1)
            g0d = pltpu.async_copy(elr_hbm.at[iedge.at[0, 1]], rdst.at[0],
                                   sem3)
            g1s = pltpu.async_copy(elr_hbm.at[iedge.at[1, 0]], rsrc.at[1],
                                   sem2)
            g1d = pltpu.async_copy(elr_hbm.at[iedge.at[1, 1]], rdst.at[1],
                                   sem4)
            g0s.wait()
            g0d.wait()
            half_compute(base, 0)
            g1s.wait()
            g1d.wait()
            half_compute(base, 1)
            for h2 in range(2):
                pltpu.make_async_copy(
                    sbuf.at[h2], s_hbm.at[pl.ds(base + h2 * C2, C2)],
                    sems[h2]).wait()
            return carry

        lax.fori_loop(0, nch_w, chunk, 0)
        plsc.subcore_barrier()

        def export_rows(off, size):
            pltpu.sync_copy(dsh.at[pl.ds(off, size)],
                            dpart_hbm.at[cid, pl.ds(off, size)])

        _per_subcore_rows(sid, N, export_rows)

    return scores


def _make_agg(N, E, H, HD):
    C2 = 128
    C = 2 * C2
    NCHG = E // C
    mesh = plsc.VectorSubcoreMesh(core_axis_name="c", subcore_axis_name="s")

    @functools.partial(
        pl.kernel,
        out_type=jax.ShapeDtypeStruct((NC, N, HD), jnp.float32),
        mesh=mesh,
        scratch_types=[
            pltpu.VMEM((2, 2, C2), jnp.int32),
            pltpu.VMEM((2, C2, H), jnp.float32),
            pltpu.VMEM((2, C2, H), jnp.float32),
            pltpu.VMEM((2, C2, H), jnp.float32),
            pltpu.VMEM((2, C2, HD), jnp.float32),
            pltpu.VMEM_SHARED((N, H), jnp.float32),
            pltpu.VMEM_SHARED((N, HD), jnp.float32),
            pltpu.SemaphoreType.DMA,
            pltpu.SemaphoreType.DMA,
            pltpu.SemaphoreType.DMA,
            pltpu.SemaphoreType.DMA,
        ],
        compiler_params=pltpu.CompilerParams(needs_layout_passes=False, use_tc_tiling_on_sc=False),
    )
    def agg(edge_hbm, fp_hbm, s_hbm, invd_hbm, z_hbm, out_hbm,
            iedge, sbuf, ibuf, abuf, fpbuf, ish, rsh,
            sem1, sem2, sem3, sem4):
        cid = lax.axis_index("c")
        sid = lax.axis_index("s")
        wid = cid * NS + sid
        nch_w = NCHG // NW + jnp.where(wid < NCHG % NW, 1, 0)

        def stage_rows(off, size):
            pltpu.sync_copy(invd_hbm.at[pl.ds(off, size)],
                            ish.at[pl.ds(off, size)])
            pltpu.sync_copy(z_hbm.at[pl.ds(off, size)],
                            rsh.at[pl.ds(off, size)])

        _per_subcore_rows(sid, N, stage_rows)
        plsc.subcore_barrier()
        iota = lax.iota(jnp.int32, L)
        cols = [jnp.full((L,), h, jnp.int32) for h in range(H)]

        def half_compute(h2):
            for g in range((C2 * H) // L):
                p = g * L + iota
                r = p >> 3
                c = p & 7
                av = (plsc.load_gather(sbuf.at[h2], [r, c])
                      * plsc.load_gather(ibuf.at[h2], [r, c]))
                plsc.store_scatter(abuf.at[h2], [r, c], av)

            def equad(e4, carry2):
                for q in range(4):
                    e = e4 * 4 + q
                    eb = jnp.full((L,), e, jnp.int32)
                    for h in range(H):
                        a = plsc.load_gather(abuf.at[h2], [eb, cols[h]])
                        v = fpbuf[h2, e, pl.ds(h * L, L)]
                        fpbuf[h2, e, pl.ds(h * L, L)] = v * a
                return carry2

            lax.fori_loop(0, C2 // 4, equad, 0)
            pltpu.sync_copy(fpbuf.at[h2], rsh.at[iedge.at[h2, 1]],
                            add=True)

        def chunk(k, carry):
            base = (wid + k * NW) * C
            e0 = pltpu.async_copy(
                edge_hbm.at[:, pl.ds(base, C2)], iedge.at[0], sem1)
            e1 = pltpu.async_copy(
                edge_hbm.at[:, pl.ds(base + C2, C2)], iedge.at[1], sem2)
            s0 = pltpu.async_copy(
                s_hbm.at[pl.ds(base, C2)], sbuf.at[0], sem3)
            s1 = pltpu.async_copy(
                s_hbm.at[pl.ds(base + C2, C2)], sbuf.at[1], sem4)
            e0.wait()
            e1.wait()
            s0.wait()
            s1.wait()
            f0 = pltpu.async_copy(fp_hbm.at[iedge.at[0, 0]], fpbuf.at[0],
                                  sem1)
            i0 = pltpu.async_copy(ish.at[iedge.at[0, 1]], ibuf.at[0],
                                  sem3)
            f1 = pltpu.async_copy(fp_hbm.at[iedge.at[1, 0]], fpbuf.at[1],
                                  sem2)
            i1 = pltpu.async_copy(ish.at[iedge.at[1, 1]], ibuf.at[1],
                                  sem4)
            i0.wait()
            f0.wait()
            half_compute(0)
            i1.wait()
            f1.wait()
            half_compute(1)
            return carry

        lax.fori_loop(0, nch_w, chunk, 0)
        plsc.subcore_barrier()

        def export_rows(off, size):
            pltpu.sync_copy(rsh.at[pl.ds(off, size)],
                            out_hbm.at[cid, pl.ds(off, size)])

        _per_subcore_rows(sid, N, export_rows)

    return agg


@jax.jit
def kernel(edge_index, feat, W, attn_l, attn_r, bias):
    N, F = feat.shape
    HD = W.shape[0]
    H = attn_l.shape[1]
    D = attn_l.shape[2]
    E = edge_index.shape[1]

    wl = (W.reshape(H, D, F) * attn_l.reshape(H, D, 1)).sum(axis=1)
    wr = (W.reshape(H, D, F) * attn_r.reshape(H, D, 1)).sum(axis=1)
    wcat = jnp.concatenate([wl, wr], axis=0)

    RB = 1000
    fp, elr = pl.pallas_call(
        _proj_body,
        grid=(N // RB,),
        in_specs=[
            pl.BlockSpec((RB, F), lambda i: (i, 0)),
            pl.BlockSpec((HD, F), lambda i: (0, 0)),
            pl.BlockSpec((2 * H, F), lambda i: (0, 0)),
        ],
        out_specs=[
            pl.BlockSpec((RB, HD), lambda i: (i, 0)),
            pl.BlockSpec((RB, 2 * H), lambda i: (i, 0)),
        ],
        out_shape=[
            jax.ShapeDtypeStruct((N, HD), jnp.float32),
            jax.ShapeDtypeStruct((N, 2 * H), jnp.float32),
        ],
    )(feat, W, wcat)

    zeros_h = jnp.zeros((N, H), jnp.float32)
    s, dparts = _make_scores(N, E, H)(edge_index, elr, zeros_h)

    invd = pl.pallas_call(
        _invd_body,
        out_shape=jax.ShapeDtypeStruct((N * H // 128, 128), jnp.float32),
    )(dparts.reshape(NC, N * H // 128, 128))
    invd = invd.reshape(N, H)

    zeros_hd = jnp.zeros((N, HD), jnp.float32)
    rparts = _make_agg(N, E, H, HD)(edge_index, fp, s, invd, zeros_hd)

    rst = pl.pallas_call(
        _merge_body,
        grid=(N // RB,),
        in_specs=[
            pl.BlockSpec((NC, RB, HD), lambda i: (0, i, 0)),
            pl.BlockSpec((1, HD), lambda i: (0, 0)),
        ],
        out_specs=pl.BlockSpec((RB, HD), lambda i: (i, 0)),
        out_shape=jax.ShapeDtypeStruct((N, HD), jnp.float32),
    )(rparts, bias.reshape(1, HD))

    return rst.reshape(N, H, D)
